# baseline (speedup 1.0000x reference)
import math

import jax
import jax.numpy as jnp
from jax.experimental import pallas as pl
from jax.experimental.pallas import tpu as pltpu

B = 2
S = 307
SP = 320
T = B * SP
D = 2048
H = 16
HD = 128
E = 8
NEG = -1e9

P1 = 3 * D // HD
P2 = B * H
FB = 256
BF = jnp.bfloat16


def _rms(x, g):
    v = jnp.mean(x * x, axis=1, keepdims=True)
    return x * jax.lax.rsqrt(v + 1e-6) * g


def _dot_t(a, b):
    return jax.lax.dot_general(a, b, (((1,), (1,)), ((), ())),
                               preferred_element_type=jnp.float32)


def _rot(x):
    return jnp.concatenate([-x[:, HD // 2:], x[:, :HD // 2]], axis=1)


def _embed_body(hist_ref, cw_ref, cb_ref, lg_ref, lb_ref, lw_ref, lb2_ref,
                o_ref, ct_ref, st_ref):
    s = hist_ref[...]
    a = jnp.maximum(s * cw_ref[...] + cb_ref[...], 0.0)
    mu = jnp.mean(a, axis=1, keepdims=True)
    var = jnp.mean((a - mu) * (a - mu), axis=1, keepdims=True)
    nrm = (a - mu) * jax.lax.rsqrt(var + 1e-5) * lg_ref[...] + lb_ref[...]
    n_last = nrm[:, -1:]
    o_ref[...] = n_last * lw_ref[...] + lb2_ref[...]
    pos = jax.lax.broadcasted_iota(jnp.int32, (SP, HD), 0).astype(jnp.float32)
    col = jax.lax.broadcasted_iota(jnp.int32, (SP, HD), 1)
    fidx = jnp.where(col >= HD // 2, col - HD // 2, col).astype(jnp.float32)
    inv = jnp.exp(fidx * (-2.0 * math.log(10000.0) / HD))
    ang = pos * inv
    ct_ref[...] = jnp.cos(ang)
    st_ref[...] = jnp.sin(ang)


def _embed(hist_pad, emb):
    full = lambda shape: pl.BlockSpec(shape, lambda *a: (0,) * len(shape))
    return pl.pallas_call(
        _embed_body,
        out_shape=[jax.ShapeDtypeStruct((T, D), jnp.float32),
                   jax.ShapeDtypeStruct((SP, HD), jnp.float32),
                   jax.ShapeDtypeStruct((SP, HD), jnp.float32)],
        in_specs=[full((T, 1))] + [full(s) for s in
                  [(1, 64), (1, 64), (1, 64), (1, 64), (1, D), (1, D)]],
        out_specs=[full((T, D)), full((SP, HD)), full((SP, HD))],
    )(hist_pad,
      emb['conv_w'].reshape(1, 64), emb['conv_b'].reshape(1, 64),
      emb['ln_g'].reshape(1, 64), emb['ln_b'].reshape(1, 64),
      emb['lin_w'].reshape(1, D), emb['lin_b'].reshape(1, D))


def _attn_phase(j, x_ref, ct_ref, st_ref, ow_ref, qkv_s, ha_s):
    j2 = j - P1
    b = j2 // H
    hh = j2 % H
    base = b * SP
    cos = ct_ref[...]
    sin = st_ref[...]
    q = qkv_s[hh, pl.ds(base, SP), :].astype(jnp.float32) * (1.0 / math.sqrt(HD))
    k = qkv_s[H + hh, pl.ds(base, SP), :].astype(jnp.float32)
    v = qkv_s[2 * H + hh, pl.ds(base, SP), :].astype(jnp.float32)
    qr = q * cos + _rot(q) * sin
    kr = k * cos + _rot(k) * sin
    sc = _dot_t(qr, kr)
    row = jax.lax.broadcasted_iota(jnp.int32, (SP, SP), 0)
    col = jax.lax.broadcasted_iota(jnp.int32, (SP, SP), 1)
    sc = sc + jnp.where(col > row, NEG, 0.0)
    m = jnp.max(sc, axis=1, keepdims=True)
    p = jnp.exp(sc - m)
    a = p * (1.0 / jnp.sum(p, axis=1, keepdims=True))
    o = jax.lax.dot_general(a, v, (((1,), (0,)), ((), ())),
                            preferred_element_type=jnp.float32)
    contrib = _dot_t(o, ow_ref[...])

    @pl.when(hh == 0)
    def _():
        ha_s[pl.ds(base, SP), :] = x_ref[pl.ds(base, SP), :] + contrib

    @pl.when(hh != 0)
    def _():
        ha_s[pl.ds(base, SP), :] = ha_s[pl.ds(base, SP), :] + contrib


def _make_dense_body(nf):
    p3 = P1 + P2

    def body(x_ref, ag_ref, fg_ref, wqkv_ref, ct_ref, st_ref, ow_ref,
             wg_ref, wu_ref, wd_ref, o_ref, qkv_s, ha_s, xn_s):
        j = pl.program_id(0)

        @pl.when(j == 0)
        def _():
            xn_s[...] = _rms(x_ref[...], ag_ref[...])

        @pl.when(j < P1)
        def _():
            qkv_s[j] = _dot_t(xn_s[...], wqkv_ref[...]).astype(BF)

        @pl.when((j >= P1) & (j < p3))
        def _():
            _attn_phase(j, x_ref, ct_ref, st_ref, ow_ref, qkv_s, ha_s)

        @pl.when(j == p3)
        def _():
            xn_s[...] = _rms(ha_s[...], fg_ref[...])
            o_ref[...] = ha_s[...]

        @pl.when(j >= p3)
        def _():
            xn = xn_s[...]
            g = _dot_t(xn, wg_ref[...])
            u = _dot_t(xn, wu_ref[...])
            a = g * jax.lax.logistic(g) * u
            o_ref[...] = o_ref[...] + _dot_t(a, wd_ref[...])

    return body


def _make_moe_body(n_sh):
    p3 = P1 + P2
    pe = p3 + n_sh

    def body(x_ref, ag_ref, fg_ref, wqkv_ref, ct_ref, st_ref, ow_ref,
             rw_ref, sg_ref, su_ref, sd_ref, eg_ref, eu_ref, ed_ref,
             o_ref, qkv_s, ha_s, xn_s, gates_s):
        j = pl.program_id(0)

        @pl.when(j == 0)
        def _():
            xn_s[...] = _rms(x_ref[...], ag_ref[...])

        @pl.when(j < P1)
        def _():
            qkv_s[j] = _dot_t(xn_s[...], wqkv_ref[...]).astype(BF)

        @pl.when((j >= P1) & (j < p3))
        def _():
            _attn_phase(j, x_ref, ct_ref, st_ref, ow_ref, qkv_s, ha_s)

        @pl.when(j == p3)
        def _():
            xn = _rms(ha_s[...], fg_ref[...])
            xn_s[...] = xn
            o_ref[...] = ha_s[...]
            logits = _dot_t(xn, rw_ref[...])
            mx = jnp.max(logits, axis=1, keepdims=True)
            ex = jnp.exp(logits - mx)
            sm = ex * (1.0 / jnp.sum(ex, axis=1, keepdims=True))
            cols = jax.lax.broadcasted_iota(jnp.int32, (T, E), 1)
            i1 = jnp.argmax(sm, axis=1)
            oh1 = cols == i1[:, None]
            m1 = jnp.max(sm, axis=1, keepdims=True)
            sm2 = jnp.where(oh1, -jnp.inf, sm)
            i2 = jnp.argmax(sm2, axis=1)
            oh2 = cols == i2[:, None]
            m2 = jnp.max(sm2, axis=1, keepdims=True)
            gates_s[...] = jnp.where(oh1, m1, 0.0) + jnp.where(oh2, m2, 0.0)

        @pl.when((j >= p3) & (j < pe))
        def _():
            xn = xn_s[...]
            g = _dot_t(xn, sg_ref[...])
            u = _dot_t(xn, su_ref[...])
            a = g * jax.lax.logistic(g) * u
            o_ref[...] = o_ref[...] + _dot_t(a, sd_ref[...])

        @pl.when(j >= pe)
        def _():
            e = (j - pe) // 2
            xn = xn_s[...]
            cols = jax.lax.broadcasted_iota(jnp.int32, (T, E), 1)
            we = jnp.sum(jnp.where(cols == e, gates_s[...], 0.0),
                         axis=1, keepdims=True)
            g = _dot_t(xn, eg_ref[0])
            u = _dot_t(xn, eu_ref[0])
            a = g * jax.lax.logistic(g) * u
            ye = _dot_t(a, ed_ref[0])
            o_ref[...] = o_ref[...] + we * ye

    return body


def _full(shape):
    return pl.BlockSpec(shape, lambda *a: (0,) * len(shape))


def _layer(h, ct, st, lp):
    is_moe = 'router_w' in lp
    p3 = P1 + P2
    wqkv = jnp.concatenate([lp['q_w'], lp['k_w'], lp['v_w']], axis=0)

    qkv_ix = lambda j: (jnp.minimum(j, P1 - 1), 0)
    ow_ix = lambda j: (0, jnp.clip(j - P1, 0, P2 - 1) % H)

    common_specs = [
        _full((T, D)), _full((1, D)), _full((1, D)),
        pl.BlockSpec((HD, D), qkv_ix),
        _full((SP, HD)), _full((SP, HD)),
        pl.BlockSpec((D, HD), ow_ix),
    ]
    common_args = [h, lp['attn_norm'].reshape(1, D), lp['ffn_norm'].reshape(1, D),
                   wqkv, ct, st, lp['o_w']]
    scratches = [pltpu.VMEM((3 * H, SP * B, HD), BF),
                 pltpu.VMEM((T, D), jnp.float32),
                 pltpu.VMEM((T, D), jnp.float32)]

    if not is_moe:
        nf = lp['gate_w'].shape[0] // FB
        f_ix = lambda j: (jnp.clip(j - p3, 0, nf - 1), 0)
        fd_ix = lambda j: (0, jnp.clip(j - p3, 0, nf - 1))
        grid = (p3 + nf,)
        body = _make_dense_body(nf)
        in_specs = common_specs + [
            pl.BlockSpec((FB, D), f_ix),
            pl.BlockSpec((FB, D), f_ix),
            pl.BlockSpec((D, FB), fd_ix),
        ]
        args = common_args + [lp['gate_w'], lp['up_w'], lp['down_w']]
    else:
        n_sh = lp['s_gate'].shape[0] // FB
        pe = p3 + n_sh
        s_ix = lambda j: (jnp.clip(j - p3, 0, n_sh - 1), 0)
        sd_ix = lambda j: (0, jnp.clip(j - p3, 0, n_sh - 1))
        e_ix = lambda j: (jnp.clip(j - pe, 0, 2 * E - 1) // 2,
                          jnp.clip(j - pe, 0, 2 * E - 1) % 2, 0)
        ed_ix = lambda j: (jnp.clip(j - pe, 0, 2 * E - 1) // 2, 0,
                           jnp.clip(j - pe, 0, 2 * E - 1) % 2)
        grid = (pe + 2 * E,)
        body = _make_moe_body(n_sh)
        in_specs = common_specs + [
            _full((E, D)),
            pl.BlockSpec((FB, D), s_ix),
            pl.BlockSpec((FB, D), s_ix),
            pl.BlockSpec((D, FB), sd_ix),
            pl.BlockSpec((1, FB, D), e_ix),
            pl.BlockSpec((1, FB, D), e_ix),
            pl.BlockSpec((1, D, FB), ed_ix),
        ]
        args = common_args + [lp['router_w'], lp['s_gate'], lp['s_up'],
                              lp['s_down'], lp['e_gate'], lp['e_up'],
                              lp['e_down']]
        scratches = scratches + [pltpu.VMEM((T, E), jnp.float32)]

    return pl.pallas_call(
        body,
        grid=grid,
        out_shape=jax.ShapeDtypeStruct((T, D), jnp.float32),
        in_specs=in_specs,
        out_specs=_full((T, D)),
        scratch_shapes=scratches,
        compiler_params=pltpu.CompilerParams(dimension_semantics=("arbitrary",)),
    )(*args)


def _head_body(x_ref, g_ref, w_ref, b_ref, o_ref):
    xn = _rms(x_ref[...], g_ref[...])
    o_ref[...] = _dot_t(xn, w_ref[...]) + b_ref[...]


def _head(x, g, w, b):
    n = w.shape[0]
    return pl.pallas_call(
        _head_body,
        out_shape=jax.ShapeDtypeStruct((T, n), jnp.float32),
        in_specs=[_full((T, D)), _full((1, D)), _full((n, D)), _full((1, n))],
        out_specs=_full((T, n)),
    )(x, g.reshape(1, D), w, b.reshape(1, n))


def kernel(history_data, emb, layers, final_norm, od_w, od_b):
    hist_pad = jnp.pad(history_data, ((0, 0), (0, SP - S))).reshape(T, 1)
    h, ct, st = _embed(hist_pad, emb)
    for lp in layers:
        h = _layer(h, ct, st, lp)
    y = _head(h, final_norm, od_w, od_b)
    return y.reshape(B, SP, S)[:, :S, :]


# speedup vs baseline: 1.1945x; 1.1945x over previous
import math

import jax
import jax.numpy as jnp
from jax.experimental import pallas as pl
from jax.experimental.pallas import tpu as pltpu

B = 2
S = 307
SP = 320
T = B * SP
D = 2048
H = 16
HD = 128
E = 8
NS = 4
NEG = -1e9


def _rms(x, g):
    v = jnp.mean(x * x, axis=1, keepdims=True)
    return x * jax.lax.rsqrt(v + 1e-6) * g


def _dot_t(a, b):
    return jax.lax.dot_general(a, b, (((1,), (1,)), ((), ())),
                               preferred_element_type=jnp.float32)


def _rot(x):
    return jnp.concatenate([-x[:, HD // 2:], x[:, :HD // 2]], axis=1)


def _embed_body(hist_ref, cw_ref, cb_ref, lg_ref, lb_ref, lw_ref, lb2_ref,
                o_ref, ct_ref, st_ref):
    s = hist_ref[...]
    a = jnp.maximum(s * cw_ref[...] + cb_ref[...], 0.0)
    mu = jnp.mean(a, axis=1, keepdims=True)
    var = jnp.mean((a - mu) * (a - mu), axis=1, keepdims=True)
    nrm = (a - mu) * jax.lax.rsqrt(var + 1e-5) * lg_ref[...] + lb_ref[...]
    n_last = nrm[:, -1:]
    o_ref[...] = n_last * lw_ref[...] + lb2_ref[...]
    pos = jax.lax.broadcasted_iota(jnp.int32, (SP, HD), 0).astype(jnp.float32)
    col = jax.lax.broadcasted_iota(jnp.int32, (SP, HD), 1)
    fidx = jnp.where(col >= HD // 2, col - HD // 2, col).astype(jnp.float32)
    inv = jnp.exp(fidx * (-2.0 * math.log(10000.0) / HD))
    ang = pos * inv
    ct_ref[...] = jnp.cos(ang)
    st_ref[...] = jnp.sin(ang)


def _rms_mm_body(x_ref, g_ref, w_ref, o_ref, xn_ref):
    @pl.when(pl.program_id(0) == 0)
    def _():
        xn_ref[...] = _rms(x_ref[...], g_ref[...])

    o_ref[...] = _dot_t(xn_ref[...], w_ref[...])


def _attn_body(qkv_ref, ct_ref, st_ref, ow_ref, r_ref, o_ref, oall_ref):
    cos = ct_ref[...]
    sin = st_ref[...]
    row = jax.lax.broadcasted_iota(jnp.int32, (SP, SP), 0)
    col = jax.lax.broadcasted_iota(jnp.int32, (SP, SP), 1)
    cmask = jnp.where(col > row, NEG, 0.0)
    for h in range(H):
        q = qkv_ref[0, :, h * HD:(h + 1) * HD] * (1.0 / math.sqrt(HD))
        k = qkv_ref[0, :, D + h * HD:D + (h + 1) * HD]
        v = qkv_ref[0, :, 2 * D + h * HD:2 * D + (h + 1) * HD]
        qr = q * cos + _rot(q) * sin
        kr = k * cos + _rot(k) * sin
        sc = _dot_t(qr, kr) + cmask
        m = jnp.max(sc, axis=1, keepdims=True)
        p = jnp.exp(sc - m)
        a = p * (1.0 / jnp.sum(p, axis=1, keepdims=True))
        oall_ref[:, h * HD:(h + 1) * HD] = jax.lax.dot_general(
            a, v, (((1,), (0,)), ((), ())), preferred_element_type=jnp.float32)
    o_ref[0] = r_ref[0] + _dot_t(oall_ref[...], ow_ref[...])


def _ffn_body(x_ref, g_ref, wg_ref, wu_ref, wd_ref, o_ref, xn_ref):
    j = pl.program_id(0)

    @pl.when(j == 0)
    def _():
        xn_ref[...] = _rms(x_ref[...], g_ref[...])

    xn = xn_ref[...]
    g = _dot_t(xn, wg_ref[...])
    u = _dot_t(xn, wu_ref[...])
    a = g * jax.lax.logistic(g) * u
    contrib = _dot_t(a, wd_ref[...])

    @pl.when(j == 0)
    def _():
        o_ref[...] = x_ref[...] + contrib

    @pl.when(j != 0)
    def _():
        o_ref[...] = o_ref[...] + contrib


def _moe_body(x_ref, g_ref, rw_ref, sg_ref, su_ref, sd_ref,
              eg_ref, eu_ref, ed_ref, o_ref, xn_ref, gates_ref):
    j = pl.program_id(0)

    @pl.when(j == 0)
    def _():
        xn = _rms(x_ref[...], g_ref[...])
        xn_ref[...] = xn
        logits = _dot_t(xn, rw_ref[...])
        mx = jnp.max(logits, axis=1, keepdims=True)
        ex = jnp.exp(logits - mx)
        sm = ex * (1.0 / jnp.sum(ex, axis=1, keepdims=True))
        cols = jax.lax.broadcasted_iota(jnp.int32, (T, E), 1)
        i1 = jnp.argmax(sm, axis=1)
        oh1 = cols == i1[:, None]
        m1 = jnp.max(sm, axis=1, keepdims=True)
        sm2 = jnp.where(oh1, -jnp.inf, sm)
        i2 = jnp.argmax(sm2, axis=1)
        oh2 = cols == i2[:, None]
        m2 = jnp.max(sm2, axis=1, keepdims=True)
        gates_ref[...] = jnp.where(oh1, m1, 0.0) + jnp.where(oh2, m2, 0.0)
        o_ref[...] = x_ref[...]

    xn = xn_ref[...]

    @pl.when(j < NS)
    def _():
        g = _dot_t(xn, sg_ref[...])
        u = _dot_t(xn, su_ref[...])
        a = g * jax.lax.logistic(g) * u
        o_ref[...] = o_ref[...] + _dot_t(a, sd_ref[...])

    @pl.when(j >= NS)
    def _():
        e = j - NS
        gates = gates_ref[...]
        cols = jax.lax.broadcasted_iota(jnp.int32, (T, E), 1)
        we = jnp.sum(jnp.where(cols == e, gates, 0.0), axis=1, keepdims=True)
        g = _dot_t(xn, eg_ref[0])
        u = _dot_t(xn, eu_ref[0])
        a = g * jax.lax.logistic(g) * u
        ye = _dot_t(a, ed_ref[0])
        o_ref[...] = o_ref[...] + we * ye


def _head_body(x_ref, g_ref, w_ref, b_ref, o_ref):
    xn = _rms(x_ref[...], g_ref[...])
    o_ref[...] = _dot_t(xn, w_ref[...]) + b_ref[...]


def _full(shape):
    return pl.BlockSpec(shape, lambda *a: (0,) * len(shape))


def _embed(hist_pad, emb):
    return pl.pallas_call(
        _embed_body,
        out_shape=[jax.ShapeDtypeStruct((T, D), jnp.float32),
                   jax.ShapeDtypeStruct((SP, HD), jnp.float32),
                   jax.ShapeDtypeStruct((SP, HD), jnp.float32)],
        in_specs=[_full((T, 1))] + [_full(s) for s in
                  [(1, 64), (1, 64), (1, 64), (1, 64), (1, D), (1, D)]],
        out_specs=[_full((T, D)), _full((SP, HD)), _full((SP, HD))],
    )(hist_pad,
      emb['conv_w'].reshape(1, 64), emb['conv_b'].reshape(1, 64),
      emb['ln_g'].reshape(1, 64), emb['ln_b'].reshape(1, 64),
      emb['lin_w'].reshape(1, D), emb['lin_b'].reshape(1, D))


def _rms_mm(x, g, w, bn=1024):
    n = w.shape[0]
    return pl.pallas_call(
        _rms_mm_body,
        grid=(n // bn,),
        out_shape=jax.ShapeDtypeStruct((T, n), jnp.float32),
        in_specs=[_full((T, D)), _full((1, D)),
                  pl.BlockSpec((bn, D), lambda j: (j, 0))],
        out_specs=pl.BlockSpec((T, bn), lambda j: (0, j)),
        scratch_shapes=[pltpu.VMEM((T, D), jnp.float32)],
        compiler_params=pltpu.CompilerParams(dimension_semantics=("arbitrary",)),
    )(x, g.reshape(1, D), w)


def _attention(qkv, ct, st, ow, r):
    return pl.pallas_call(
        _attn_body,
        grid=(B,),
        out_shape=jax.ShapeDtypeStruct((B, SP, D), jnp.float32),
        in_specs=[pl.BlockSpec((1, SP, 3 * D), lambda b: (b, 0, 0)),
                  _full((SP, HD)), _full((SP, HD)),
                  _full((D, D)),
                  pl.BlockSpec((1, SP, D), lambda b: (b, 0, 0))],
        out_specs=pl.BlockSpec((1, SP, D), lambda b: (b, 0, 0)),
        scratch_shapes=[pltpu.VMEM((SP, D), jnp.float32)],
        compiler_params=pltpu.CompilerParams(dimension_semantics=("arbitrary",)),
    )(qkv, ct, st, ow, r)


def _ffn(x, g, wg, wu, wd, bf=512):
    ff = wg.shape[0]
    return pl.pallas_call(
        _ffn_body,
        grid=(ff // bf,),
        out_shape=jax.ShapeDtypeStruct((T, D), jnp.float32),
        in_specs=[_full((T, D)), _full((1, D)),
                  pl.BlockSpec((bf, D), lambda j: (j, 0)),
                  pl.BlockSpec((bf, D), lambda j: (j, 0)),
                  pl.BlockSpec((D, bf), lambda j: (0, j))],
        out_specs=_full((T, D)),
        scratch_shapes=[pltpu.VMEM((T, D), jnp.float32)],
        compiler_params=pltpu.CompilerParams(dimension_semantics=("arbitrary",)),
    )(x, g.reshape(1, D), wg, wu, wd)


def _moe(x, g, rw, sg, su, sd, eg, eu, ed):
    mf = eg.shape[1]
    sb = sg.shape[0] // NS
    eix = lambda j: (jnp.maximum(j - NS, 0), 0, 0)
    six = lambda j: (jnp.minimum(j, NS - 1), 0)
    return pl.pallas_call(
        _moe_body,
        grid=(NS + E,),
        out_shape=jax.ShapeDtypeStruct((T, D), jnp.float32),
        in_specs=[_full((T, D)), _full((1, D)), _full((E, D)),
                  pl.BlockSpec((sb, D), six),
                  pl.BlockSpec((sb, D), six),
                  pl.BlockSpec((D, sb), lambda j: (0, jnp.minimum(j, NS - 1))),
                  pl.BlockSpec((1, mf, D), eix),
                  pl.BlockSpec((1, mf, D), eix),
                  pl.BlockSpec((1, D, mf), eix)],
        out_specs=_full((T, D)),
        scratch_shapes=[pltpu.VMEM((T, D), jnp.float32),
                        pltpu.VMEM((T, E), jnp.float32)],
        compiler_params=pltpu.CompilerParams(dimension_semantics=("arbitrary",)),
    )(x, g.reshape(1, D), rw, sg, su, sd, eg, eu, ed)


def _head(x, g, w, b):
    n = w.shape[0]
    return pl.pallas_call(
        _head_body,
        out_shape=jax.ShapeDtypeStruct((T, n), jnp.float32),
        in_specs=[_full((T, D)), _full((1, D)), _full((n, D)), _full((1, n))],
        out_specs=_full((T, n)),
    )(x, g.reshape(1, D), w, b.reshape(1, n))


def kernel(history_data, emb, layers, final_norm, od_w, od_b):
    hist_pad = jnp.pad(history_data, ((0, 0), (0, SP - S))).reshape(T, 1)
    h, ct, st = _embed(hist_pad, emb)

    for lp in layers:
        wqkv = jnp.concatenate([lp['q_w'], lp['k_w'], lp['v_w']], axis=0)
        qkv = _rms_mm(h, lp['attn_norm'], wqkv)
        h = _attention(qkv.reshape(B, SP, 3 * D), ct, st,
                       lp['o_w'], h.reshape(B, SP, D)).reshape(T, D)
        if 'router_w' in lp:
            h = _moe(h, lp['ffn_norm'], lp['router_w'],
                     lp['s_gate'], lp['s_up'], lp['s_down'],
                     lp['e_gate'], lp['e_up'], lp['e_down'])
        else:
            h = _ffn(h, lp['ffn_norm'], lp['gate_w'], lp['up_w'], lp['down_w'])

    y = _head(h, final_norm, od_w, od_b)
    return y.reshape(B, SP, S)[:, :S, :]


# speedup vs baseline: 1.3943x; 1.1673x over previous
import math

import jax
import jax.numpy as jnp
from jax.experimental import pallas as pl
from jax.experimental.pallas import tpu as pltpu

B = 2
S = 307
SP = 320
T = B * SP
D = 2048
H = 16
HD = 128
E = 8
NS = 4
NEG = -1e9


def _rms(x, g):
    v = jnp.mean(x * x, axis=1, keepdims=True)
    return x * jax.lax.rsqrt(v + 1e-6) * g


def _dot_t(a, b):
    return jax.lax.dot_general(a, b, (((1,), (1,)), ((), ())),
                               preferred_element_type=jnp.float32)


def _rot(x):
    return jnp.concatenate([-x[:, HD // 2:], x[:, :HD // 2]], axis=1)


def _embed_body(hist_ref, cw_ref, cb_ref, lg_ref, lb_ref, lw_ref, lb2_ref,
                o_ref, ct_ref, st_ref):
    s = hist_ref[...]
    a = jnp.maximum(s * cw_ref[...] + cb_ref[...], 0.0)
    mu = jnp.mean(a, axis=1, keepdims=True)
    var = jnp.mean((a - mu) * (a - mu), axis=1, keepdims=True)
    nrm = (a - mu) * jax.lax.rsqrt(var + 1e-5) * lg_ref[...] + lb_ref[...]
    n_last = nrm[:, -1:]
    o_ref[...] = n_last * lw_ref[...] + lb2_ref[...]
    pos = jax.lax.broadcasted_iota(jnp.int32, (SP, HD), 0).astype(jnp.float32)
    col = jax.lax.broadcasted_iota(jnp.int32, (SP, HD), 1)
    fidx = jnp.where(col >= HD // 2, col - HD // 2, col).astype(jnp.float32)
    inv = jnp.exp(fidx * (-2.0 * math.log(10000.0) / HD))
    ang = pos * inv
    ct_ref[...] = jnp.cos(ang)
    st_ref[...] = jnp.sin(ang)


def _qkv_body(x_ref, g_ref, wq_ref, wk_ref, wv_ref, o_ref, xn_ref):
    j = pl.program_id(0)

    @pl.when(j == 0)
    def _():
        xn_ref[...] = _rms(x_ref[...], g_ref[...])

    xn = xn_ref[...]
    nb = pl.num_programs(0) // 3

    @pl.when(j < nb)
    def _():
        o_ref[...] = _dot_t(xn, wq_ref[...])

    @pl.when((j >= nb) & (j < 2 * nb))
    def _():
        o_ref[...] = _dot_t(xn, wk_ref[...])

    @pl.when(j >= 2 * nb)
    def _():
        o_ref[...] = _dot_t(xn, wv_ref[...])


def _attn_body(qkv_ref, ct_ref, st_ref, ow_ref, r_ref, o_ref, oall_ref):
    cos = ct_ref[...]
    sin = st_ref[...]
    row = jax.lax.broadcasted_iota(jnp.int32, (SP, SP), 0)
    col = jax.lax.broadcasted_iota(jnp.int32, (SP, SP), 1)
    cmask = jnp.where(col > row, NEG, 0.0)
    for h in range(H):
        q = qkv_ref[0, :, h * HD:(h + 1) * HD] * (1.0 / math.sqrt(HD))
        k = qkv_ref[0, :, D + h * HD:D + (h + 1) * HD]
        v = qkv_ref[0, :, 2 * D + h * HD:2 * D + (h + 1) * HD]
        qr = q * cos + _rot(q) * sin
        kr = k * cos + _rot(k) * sin
        sc = _dot_t(qr, kr) + cmask
        m = jnp.max(sc, axis=1, keepdims=True)
        p = jnp.exp(sc - m)
        a = p * (1.0 / jnp.sum(p, axis=1, keepdims=True))
        oall_ref[:, h * HD:(h + 1) * HD] = jax.lax.dot_general(
            a, v, (((1,), (0,)), ((), ())), preferred_element_type=jnp.float32)
    o_ref[0] = r_ref[0] + _dot_t(oall_ref[...], ow_ref[...])


def _ffn_body(x_ref, g_ref, wg_ref, wu_ref, wd_ref, o_ref, xn_ref):
    j = pl.program_id(0)

    @pl.when(j == 0)
    def _():
        xn_ref[...] = _rms(x_ref[...], g_ref[...])

    xn = xn_ref[...]
    g = _dot_t(xn, wg_ref[...])
    u = _dot_t(xn, wu_ref[...])
    a = g * jax.lax.logistic(g) * u
    contrib = _dot_t(a, wd_ref[...])

    @pl.when(j == 0)
    def _():
        o_ref[...] = x_ref[...] + contrib

    @pl.when(j != 0)
    def _():
        o_ref[...] = o_ref[...] + contrib


def _moe_body(x_ref, g_ref, rw_ref, sg_ref, su_ref, sd_ref,
              eg_ref, eu_ref, ed_ref, o_ref, xn_ref, gates_ref):
    j = pl.program_id(0)

    @pl.when(j == 0)
    def _():
        xn = _rms(x_ref[...], g_ref[...])
        xn_ref[...] = xn
        logits = _dot_t(xn, rw_ref[...])
        mx = jnp.max(logits, axis=1, keepdims=True)
        ex = jnp.exp(logits - mx)
        sm = ex * (1.0 / jnp.sum(ex, axis=1, keepdims=True))
        cols = jax.lax.broadcasted_iota(jnp.int32, (T, E), 1)
        i1 = jnp.argmax(sm, axis=1)
        oh1 = cols == i1[:, None]
        m1 = jnp.max(sm, axis=1, keepdims=True)
        sm2 = jnp.where(oh1, -jnp.inf, sm)
        i2 = jnp.argmax(sm2, axis=1)
        oh2 = cols == i2[:, None]
        m2 = jnp.max(sm2, axis=1, keepdims=True)
        gates_ref[...] = jnp.where(oh1, m1, 0.0) + jnp.where(oh2, m2, 0.0)
        o_ref[...] = x_ref[...]

    xn = xn_ref[...]

    @pl.when(j < NS)
    def _():
        g = _dot_t(xn, sg_ref[...])
        u = _dot_t(xn, su_ref[...])
        a = g * jax.lax.logistic(g) * u
        o_ref[...] = o_ref[...] + _dot_t(a, sd_ref[...])

    @pl.when(j >= NS)
    def _():
        e = j - NS
        gates = gates_ref[...]
        cols = jax.lax.broadcasted_iota(jnp.int32, (T, E), 1)
        we = jnp.sum(jnp.where(cols == e, gates, 0.0), axis=1, keepdims=True)
        g = _dot_t(xn, eg_ref[0])
        u = _dot_t(xn, eu_ref[0])
        a = g * jax.lax.logistic(g) * u
        ye = _dot_t(a, ed_ref[0])
        o_ref[...] = o_ref[...] + we * ye


def _head_body(x_ref, g_ref, w_ref, b_ref, o_ref):
    xn = _rms(x_ref[...], g_ref[...])
    o_ref[...] = _dot_t(xn, w_ref[...]) + b_ref[...]


def _full(shape):
    return pl.BlockSpec(shape, lambda *a: (0,) * len(shape))


def _embed(hist_pad, emb):
    return pl.pallas_call(
        _embed_body,
        out_shape=[jax.ShapeDtypeStruct((T, D), jnp.float32),
                   jax.ShapeDtypeStruct((SP, HD), jnp.float32),
                   jax.ShapeDtypeStruct((SP, HD), jnp.float32)],
        in_specs=[_full((T, 1))] + [_full(s) for s in
                  [(1, 64), (1, 64), (1, 64), (1, 64), (1, D), (1, D)]],
        out_specs=[_full((T, D)), _full((SP, HD)), _full((SP, HD))],
    )(hist_pad,
      emb['conv_w'].reshape(1, 64), emb['conv_b'].reshape(1, 64),
      emb['ln_g'].reshape(1, 64), emb['ln_b'].reshape(1, 64),
      emb['lin_w'].reshape(1, D), emb['lin_b'].reshape(1, D))


def _qkv_mm(x, g, wq, wk, wv, bn=512):
    nb = D // bn
    wix = lambda off: (lambda j: (jnp.clip(j - off, 0, nb - 1), 0))
    return pl.pallas_call(
        _qkv_body,
        grid=(3 * nb,),
        out_shape=jax.ShapeDtypeStruct((T, 3 * D), jnp.float32),
        in_specs=[_full((T, D)), _full((1, D)),
                  pl.BlockSpec((bn, D), wix(0)),
                  pl.BlockSpec((bn, D), wix(nb)),
                  pl.BlockSpec((bn, D), wix(2 * nb))],
        out_specs=pl.BlockSpec((T, bn), lambda j: (0, j)),
        scratch_shapes=[pltpu.VMEM((T, D), jnp.float32)],
        compiler_params=pltpu.CompilerParams(dimension_semantics=("arbitrary",)),
    )(x, g.reshape(1, D), wq, wk, wv)


def _attention(qkv, ct, st, ow, r):
    return pl.pallas_call(
        _attn_body,
        grid=(B,),
        out_shape=jax.ShapeDtypeStruct((B, SP, D), jnp.float32),
        in_specs=[pl.BlockSpec((1, SP, 3 * D), lambda b: (b, 0, 0)),
                  _full((SP, HD)), _full((SP, HD)),
                  _full((D, D)),
                  pl.BlockSpec((1, SP, D), lambda b: (b, 0, 0))],
        out_specs=pl.BlockSpec((1, SP, D), lambda b: (b, 0, 0)),
        scratch_shapes=[pltpu.VMEM((SP, D), jnp.float32)],
        compiler_params=pltpu.CompilerParams(dimension_semantics=("arbitrary",)),
    )(qkv, ct, st, ow, r)


def _ffn(x, g, wg, wu, wd, bf=512):
    ff = wg.shape[0]
    return pl.pallas_call(
        _ffn_body,
        grid=(ff // bf,),
        out_shape=jax.ShapeDtypeStruct((T, D), jnp.float32),
        in_specs=[_full((T, D)), _full((1, D)),
                  pl.BlockSpec((bf, D), lambda j: (j, 0)),
                  pl.BlockSpec((bf, D), lambda j: (j, 0)),
                  pl.BlockSpec((D, bf), lambda j: (0, j))],
        out_specs=_full((T, D)),
        scratch_shapes=[pltpu.VMEM((T, D), jnp.float32)],
        compiler_params=pltpu.CompilerParams(dimension_semantics=("arbitrary",)),
    )(x, g.reshape(1, D), wg, wu, wd)


def _moe(x, g, rw, sg, su, sd, eg, eu, ed):
    mf = eg.shape[1]
    sb = sg.shape[0] // NS
    eix = lambda j: (jnp.maximum(j - NS, 0), 0, 0)
    six = lambda j: (jnp.minimum(j, NS - 1), 0)
    return pl.pallas_call(
        _moe_body,
        grid=(NS + E,),
        out_shape=jax.ShapeDtypeStruct((T, D), jnp.float32),
        in_specs=[_full((T, D)), _full((1, D)), _full((E, D)),
                  pl.BlockSpec((sb, D), six),
                  pl.BlockSpec((sb, D), six),
                  pl.BlockSpec((D, sb), lambda j: (0, jnp.minimum(j, NS - 1))),
                  pl.BlockSpec((1, mf, D), eix),
                  pl.BlockSpec((1, mf, D), eix),
                  pl.BlockSpec((1, D, mf), eix)],
        out_specs=_full((T, D)),
        scratch_shapes=[pltpu.VMEM((T, D), jnp.float32),
                        pltpu.VMEM((T, E), jnp.float32)],
        compiler_params=pltpu.CompilerParams(dimension_semantics=("arbitrary",)),
    )(x, g.reshape(1, D), rw, sg, su, sd, eg, eu, ed)


def _head(x, g, w, b):
    n = w.shape[0]
    return pl.pallas_call(
        _head_body,
        out_shape=jax.ShapeDtypeStruct((T, n), jnp.float32),
        in_specs=[_full((T, D)), _full((1, D)), _full((n, D)), _full((1, n))],
        out_specs=_full((T, n)),
    )(x, g.reshape(1, D), w, b.reshape(1, n))


def kernel(history_data, emb, layers, final_norm, od_w, od_b):
    hist_pad = jnp.pad(history_data, ((0, 0), (0, SP - S))).reshape(T, 1)
    h, ct, st = _embed(hist_pad, emb)

    for lp in layers:
        qkv = _qkv_mm(h, lp['attn_norm'], lp['q_w'], lp['k_w'], lp['v_w'])
        h = _attention(qkv.reshape(B, SP, 3 * D), ct, st,
                       lp['o_w'], h.reshape(B, SP, D)).reshape(T, D)
        if 'router_w' in lp:
            h = _moe(h, lp['ffn_norm'], lp['router_w'],
                     lp['s_gate'], lp['s_up'], lp['s_down'],
                     lp['e_gate'], lp['e_up'], lp['e_down'])
        else:
            h = _ffn(h, lp['ffn_norm'], lp['gate_w'], lp['up_w'], lp['down_w'])

    y = _head(h, final_norm, od_w, od_b)
    return y.reshape(B, SP, S)[:, :S, :]
